# baseline (speedup 1.0000x reference)
import functools

import jax
import jax.numpy as jnp
from jax import lax
from jax.experimental import pallas as pl
from jax.experimental.pallas import tpu as pltpu
from jax.experimental.pallas import tpu_sc as plsc

N_NODES = 10000
N_EDGES = 320000
D = 128

NC = 2
NS = 16
NW = NC * NS
CHUNK = 128
RPW = 80
RPH = RPW // 2
CROWS = NW * RPW
EPAD = CROWS * CHUNK
NPAD = 10240
ROWS_PER_TILE = NPAD // NS


def _sc_aggregate(x, src2, dst2, zrows):
    mesh = plsc.VectorSubcoreMesh(core_axis_name="c", subcore_axis_name="s")

    @functools.partial(
        pl.kernel,
        mesh=mesh,
        out_type=jax.ShapeDtypeStruct((NC, NPAD, D), jnp.float32),
        scratch_types=[
            pltpu.VMEM((RPH, CHUNK), jnp.int32),
            pltpu.VMEM((RPH, CHUNK), jnp.int32),
            pltpu.VMEM((CHUNK, D), jnp.float32),
            pltpu.VMEM((CHUNK, D), jnp.float32),
            pltpu.VMEM_SHARED((NPAD, D), jnp.float32),
            pltpu.SemaphoreType.DMA,
            pltpu.SemaphoreType.DMA,
        ],
    )
    def agg_kernel(x_hbm, src_hbm, dst_hbm, z_hbm, out_hbm,
                   srcv, dstv, rows_a, rows_b, agg_sh, sem_a, sem_b):
        cid = lax.axis_index("c")
        sid = lax.axis_index("s")
        wid = sid * NC + cid

        pltpu.sync_copy(z_hbm,
                        agg_sh.at[pl.ds(sid * ROWS_PER_TILE, ROWS_PER_TILE)])
        plsc.subcore_barrier()

        rbase = wid * RPW
        for h in range(RPW // RPH):
            hbase = rbase + h * RPH
            pltpu.sync_copy(src_hbm.at[pl.ds(hbase, RPH)], srcv)
            pltpu.sync_copy(dst_hbm.at[pl.ds(hbase, RPH)], dstv)

            def body(k, carry):
                pltpu.async_copy(x_hbm.at[srcv.at[k]], rows_a, sem_a).wait()
                pltpu.sync_copy(rows_a, agg_sh.at[dstv.at[k]], add=True)
                return carry

            lax.fori_loop(0, RPH, body, 0)

        plsc.subcore_barrier()
        pltpu.sync_copy(
            agg_sh.at[pl.ds(sid * ROWS_PER_TILE, ROWS_PER_TILE)],
            out_hbm.at[cid, pl.ds(sid * ROWS_PER_TILE, ROWS_PER_TILE)])

    return agg_kernel(x, src2, dst2, zrows)


BN = 2000


def _tc_finish(parts, x, W):
    def body(p_ref, x_ref, w_ref, o_ref):
        agg = p_ref[0] + p_ref[1]
        h = lax.dot_general(agg, w_ref[...], (((1,), (1,)), ((), ())),
                            preferred_element_type=jnp.float32)
        o_ref[...] = jnp.maximum(h, 0.0) + x_ref[...]

    return pl.pallas_call(
        body,
        grid=(N_NODES // BN,),
        in_specs=[
            pl.BlockSpec((NC, BN, D), lambda i: (0, i, 0)),
            pl.BlockSpec((BN, D), lambda i: (i, 0)),
            pl.BlockSpec((D, D), lambda i: (0, 0)),
        ],
        out_specs=pl.BlockSpec((BN, D), lambda i: (i, 0)),
        out_shape=jax.ShapeDtypeStruct((N_NODES, D), jnp.float32),
    )(parts, x, W)


def kernel(x, edge_index, W):
    src = edge_index[0].astype(jnp.int32)
    dst = edge_index[1].astype(jnp.int32)
    pad = EPAD - N_EDGES
    src2 = jnp.concatenate(
        [src, jnp.zeros((pad,), jnp.int32)]).reshape(CROWS, CHUNK)
    dst2 = jnp.concatenate(
        [dst, N_NODES + (jnp.arange(pad, dtype=jnp.int32) % (NPAD - N_NODES))]
    ).reshape(CROWS, CHUNK)
    zrows = jnp.zeros((ROWS_PER_TILE, D), jnp.float32)
    parts = _sc_aggregate(x, src2, dst2, zrows)
    return _tc_finish(parts, x, W)


# speedup vs baseline: 1.0505x; 1.0505x over previous
import functools

import jax
import jax.numpy as jnp
from jax import lax
from jax.experimental import pallas as pl
from jax.experimental.pallas import tpu as pltpu
from jax.experimental.pallas import tpu_sc as plsc

N_NODES = 10000
N_EDGES = 320000
D = 128

NC = 2
NS = 16
NW = NC * NS
CHUNK = 128
RPW = 80
EPW = RPW * CHUNK
EPAD = NW * EPW
EXTRA = 2 * CHUNK
NPAD = 10240
ROWS_PER_TILE = NPAD // NS


def _sc_aggregate(x, src, dst, zrows):
    mesh = plsc.VectorSubcoreMesh(core_axis_name="c", subcore_axis_name="s")

    @functools.partial(
        pl.kernel,
        mesh=mesh,
        out_type=jax.ShapeDtypeStruct((NC, NPAD, D), jnp.float32),
        scratch_types=[
            pltpu.VMEM((CHUNK,), jnp.int32),
            pltpu.VMEM((CHUNK,), jnp.int32),
            pltpu.VMEM((CHUNK,), jnp.int32),
            pltpu.VMEM((CHUNK,), jnp.int32),
            pltpu.VMEM((CHUNK, D), jnp.float32),
            pltpu.VMEM((CHUNK, D), jnp.float32),
            pltpu.VMEM_SHARED((NPAD, D), jnp.float32),
            pltpu.SemaphoreType.DMA,
            pltpu.SemaphoreType.DMA,
            pltpu.SemaphoreType.DMA,
            pltpu.SemaphoreType.DMA,
        ],
    )
    def agg_kernel(x_hbm, src_hbm, dst_hbm, z_hbm, out_hbm,
                   srcv_a, dstv_a, srcv_b, dstv_b, rows_a, rows_b, agg_sh,
                   semi_a, semi_b, semr_a, semr_b):
        cid = lax.axis_index("c")
        sid = lax.axis_index("s")
        wid = sid * NC + cid
        ebase = wid * EPW

        def load_idx(j, srcv, dstv, semi):
            base = ebase + j * CHUNK
            pltpu.async_copy(src_hbm.at[pl.ds(base, CHUNK)], srcv, semi)
            pltpu.async_copy(dst_hbm.at[pl.ds(base, CHUNK)], dstv, semi)

        def wait_idx(j, srcv, dstv, semi):
            base = ebase + j * CHUNK
            pltpu.make_async_copy(
                src_hbm.at[pl.ds(base, CHUNK)], srcv, semi).wait()
            pltpu.make_async_copy(
                dst_hbm.at[pl.ds(base, CHUNK)], dstv, semi).wait()

        pltpu.sync_copy(z_hbm,
                        agg_sh.at[pl.ds(sid * ROWS_PER_TILE, ROWS_PER_TILE)])
        plsc.subcore_barrier()

        load_idx(0, srcv_a, dstv_a, semi_a)
        wait_idx(0, srcv_a, dstv_a, semi_a)
        pltpu.async_copy(x_hbm.at[srcv_a], rows_a, semr_a)
        load_idx(1, srcv_b, dstv_b, semi_b)

        def body(k, carry):
            j0 = 2 * k
            j1 = j0 + 1
            wait_idx(j1, srcv_b, dstv_b, semi_b)
            pltpu.async_copy(x_hbm.at[srcv_b], rows_b, semr_b)
            pltpu.make_async_copy(x_hbm.at[srcv_a], rows_a, semr_a).wait()
            pltpu.sync_copy(rows_a, agg_sh.at[dstv_a], add=True)
            load_idx(j0 + 2, srcv_a, dstv_a, semi_a)
            pltpu.make_async_copy(x_hbm.at[srcv_b], rows_b, semr_b).wait()
            pltpu.sync_copy(rows_b, agg_sh.at[dstv_b], add=True)
            wait_idx(j0 + 2, srcv_a, dstv_a, semi_a)
            pltpu.async_copy(x_hbm.at[srcv_a], rows_a, semr_a)
            load_idx(j1 + 2, srcv_b, dstv_b, semi_b)
            return carry

        lax.fori_loop(0, RPW // 2, body, 0)

        pltpu.make_async_copy(x_hbm.at[srcv_a], rows_a, semr_a).wait()
        wait_idx(RPW + 1, srcv_b, dstv_b, semi_b)

        plsc.subcore_barrier()
        pltpu.sync_copy(
            agg_sh.at[pl.ds(sid * ROWS_PER_TILE, ROWS_PER_TILE)],
            out_hbm.at[cid, pl.ds(sid * ROWS_PER_TILE, ROWS_PER_TILE)])

    return agg_kernel(x, src, dst, zrows)


BN = 2000


def _tc_finish(parts, x, W):
    def body(p_ref, x_ref, w_ref, o_ref):
        agg = p_ref[0] + p_ref[1]
        h = lax.dot_general(agg, w_ref[...], (((1,), (1,)), ((), ())),
                            preferred_element_type=jnp.float32)
        o_ref[...] = jnp.maximum(h, 0.0) + x_ref[...]

    return pl.pallas_call(
        body,
        grid=(N_NODES // BN,),
        in_specs=[
            pl.BlockSpec((NC, BN, D), lambda i: (0, i, 0)),
            pl.BlockSpec((BN, D), lambda i: (i, 0)),
            pl.BlockSpec((D, D), lambda i: (0, 0)),
        ],
        out_specs=pl.BlockSpec((BN, D), lambda i: (i, 0)),
        out_shape=jax.ShapeDtypeStruct((N_NODES, D), jnp.float32),
    )(parts, x, W)


def kernel(x, edge_index, W):
    src = edge_index[0].astype(jnp.int32)
    dst = edge_index[1].astype(jnp.int32)
    pad = EPAD + EXTRA - N_EDGES
    src_p = jnp.concatenate([src, jnp.zeros((pad,), jnp.int32)])
    dst_p = jnp.concatenate(
        [dst, N_NODES + (jnp.arange(pad, dtype=jnp.int32) % (NPAD - N_NODES))])
    zrows = jnp.zeros((ROWS_PER_TILE, D), jnp.float32)
    parts = _sc_aggregate(x, src_p, dst_p, zrows)
    return _tc_finish(parts, x, W)


# speedup vs baseline: 3.3556x; 3.1944x over previous
import functools

import jax
import jax.numpy as jnp
from jax import lax
from jax.experimental import pallas as pl
from jax.experimental.pallas import tpu as pltpu
from jax.experimental.pallas import tpu_sc as plsc

N_NODES = 10000
N_EDGES = 320000
D = 128

NC = 2
NS = 16
NW = NC * NS
CHUNK = 128
RPW = 80
EPW = RPW * CHUNK
EPAD = NW * EPW
EXTRA = 2 * CHUNK
NPAD = 10240
ROWS_PER_TILE = NPAD // NS


def _sc_aggregate(x, src, dst, zrows):
    mesh = plsc.VectorSubcoreMesh(core_axis_name="c", subcore_axis_name="s")

    @functools.partial(
        pl.kernel,
        mesh=mesh,
        out_type=jax.ShapeDtypeStruct((NC, NPAD, D), jnp.float32),
        scratch_types=[
            pltpu.VMEM((CHUNK,), jnp.int32),
            pltpu.VMEM((CHUNK,), jnp.int32),
            pltpu.VMEM((CHUNK,), jnp.int32),
            pltpu.VMEM((CHUNK,), jnp.int32),
            pltpu.VMEM((CHUNK, D), jnp.float32),
            pltpu.VMEM((CHUNK, D), jnp.float32),
            pltpu.VMEM_SHARED((NPAD, D), jnp.float32),
            pltpu.SemaphoreType.DMA,
            pltpu.SemaphoreType.DMA,
            pltpu.SemaphoreType.DMA,
            pltpu.SemaphoreType.DMA,
        ],
    )
    def agg_kernel(x_hbm, src_hbm, dst_hbm, z_hbm, out_hbm,
                   srcv_a, dstv_a, srcv_b, dstv_b, rows_a, rows_b, agg_sh,
                   semi_a, semi_b, semr_a, semr_b):
        cid = lax.axis_index("c")
        sid = lax.axis_index("s")
        wid = sid * NC + cid
        ebase = wid * EPW

        def load_idx(j, srcv, dstv, semi):
            base = ebase + j * CHUNK
            pltpu.async_copy(src_hbm.at[pl.ds(base, CHUNK)], srcv, semi)
            pltpu.async_copy(dst_hbm.at[pl.ds(base, CHUNK)], dstv, semi)

        def wait_idx(j, srcv, dstv, semi):
            base = ebase + j * CHUNK
            pltpu.make_async_copy(
                src_hbm.at[pl.ds(base, CHUNK)], srcv, semi).wait()
            pltpu.make_async_copy(
                dst_hbm.at[pl.ds(base, CHUNK)], dstv, semi).wait()

        pltpu.sync_copy(z_hbm,
                        agg_sh.at[pl.ds(sid * ROWS_PER_TILE, ROWS_PER_TILE)])
        plsc.subcore_barrier()

        load_idx(0, srcv_a, dstv_a, semi_a)
        wait_idx(0, srcv_a, dstv_a, semi_a)
        pltpu.async_copy(x_hbm.at[srcv_a], rows_a, semr_a)
        load_idx(1, srcv_b, dstv_b, semi_b)

        def body(k, carry):
            j0 = 2 * k
            j1 = j0 + 1
            wait_idx(j1, srcv_b, dstv_b, semi_b)
            pltpu.async_copy(x_hbm.at[srcv_b], rows_b, semr_b)
            pltpu.make_async_copy(x_hbm.at[srcv_a], rows_a, semr_a).wait()
            pltpu.sync_copy(rows_a, agg_sh.at[dstv_a], add=True)
            load_idx(j0 + 2, srcv_a, dstv_a, semi_a)
            pltpu.make_async_copy(x_hbm.at[srcv_b], rows_b, semr_b).wait()
            pltpu.sync_copy(rows_b, agg_sh.at[dstv_b], add=True)
            wait_idx(j0 + 2, srcv_a, dstv_a, semi_a)
            pltpu.async_copy(x_hbm.at[srcv_a], rows_a, semr_a)
            load_idx(j1 + 2, srcv_b, dstv_b, semi_b)
            return carry

        lax.fori_loop(0, RPW // 2, body, 0)

        pltpu.make_async_copy(x_hbm.at[srcv_a], rows_a, semr_a).wait()
        wait_idx(RPW + 1, srcv_b, dstv_b, semi_b)

        plsc.subcore_barrier()
        pltpu.sync_copy(
            agg_sh.at[pl.ds(sid * ROWS_PER_TILE, ROWS_PER_TILE)],
            out_hbm.at[cid, pl.ds(sid * ROWS_PER_TILE, ROWS_PER_TILE)])

    return agg_kernel(x, src, dst, zrows)


BN = 2000


def _tc_finish(parts, x, W):
    def body(p_ref, x_ref, w_ref, o_ref):
        agg = p_ref[0] + p_ref[1]
        h = lax.dot_general(agg, w_ref[...], (((1,), (1,)), ((), ())),
                            preferred_element_type=jnp.float32)
        o_ref[...] = jnp.maximum(h, 0.0) + x_ref[...]

    return pl.pallas_call(
        body,
        grid=(N_NODES // BN,),
        in_specs=[
            pl.BlockSpec((NC, BN, D), lambda i: (0, i, 0)),
            pl.BlockSpec((BN, D), lambda i: (i, 0)),
            pl.BlockSpec((D, D), lambda i: (0, 0)),
        ],
        out_specs=pl.BlockSpec((BN, D), lambda i: (i, 0)),
        out_shape=jax.ShapeDtypeStruct((N_NODES, D), jnp.float32),
    )(parts, x, W)


def kernel(x, edge_index, W):
    src = edge_index[0].astype(jnp.int32)
    dst = edge_index[1].astype(jnp.int32)
    pad = EPAD + EXTRA - N_EDGES
    arange_pad = jnp.arange(pad, dtype=jnp.int32)
    src_p = jnp.concatenate([src, N_NODES + arange_pad % (NPAD - N_NODES)])
    dst_p = jnp.concatenate([dst, arange_pad % NPAD])
    x_g = jnp.concatenate(
        [x, jnp.zeros((NPAD - N_NODES, D), jnp.float32)])
    zrows = jnp.zeros((ROWS_PER_TILE, D), jnp.float32)
    parts = _sc_aggregate(x_g, src_p, dst_p, zrows)
    return _tc_finish(parts, x, W)


# speedup vs baseline: 3.9091x; 1.1650x over previous
import functools

import jax
import jax.numpy as jnp
from jax import lax
from jax.experimental import pallas as pl
from jax.experimental.pallas import tpu as pltpu
from jax.experimental.pallas import tpu_sc as plsc

N_NODES = 10000
N_EDGES = 320000
D = 128

NC = 2
NS = 16
NW = NC * NS
CHUNK = 128
RPW = 80
RPH = RPW // 2
EPW = RPW * CHUNK
EPAD = NW * EPW
EXTRA = 0
NPAD = 10240
ROWS_PER_TILE = NPAD // NS


def _sc_aggregate(x, src, dst, zrows):
    mesh = plsc.VectorSubcoreMesh(core_axis_name="c", subcore_axis_name="s")

    @functools.partial(
        pl.kernel,
        mesh=mesh,
        out_type=jax.ShapeDtypeStruct((NC, NPAD, D), jnp.float32),
        scratch_types=[
            pltpu.VMEM((RPH, CHUNK), jnp.int32),
            pltpu.VMEM((RPH, CHUNK), jnp.int32),
            pltpu.VMEM((CHUNK, D), jnp.float32),
            pltpu.VMEM((CHUNK, D), jnp.float32),
            pltpu.VMEM_SHARED((NPAD, D), jnp.float32),
            pltpu.SemaphoreType.DMA,
            pltpu.SemaphoreType.DMA,
        ],
    )
    def agg_kernel(x_hbm, src_hbm, dst_hbm, z_hbm, out_hbm,
                   srcv, dstv, rows_a, rows_b, agg_sh, sem_a, sem_b):
        cid = lax.axis_index("c")
        sid = lax.axis_index("s")
        wid = sid * NC + cid

        pltpu.sync_copy(z_hbm,
                        agg_sh.at[pl.ds(sid * ROWS_PER_TILE, ROWS_PER_TILE)])
        plsc.subcore_barrier()

        rbase = wid * RPW
        for h in range(RPW // RPH):
            hbase = rbase + h * RPH
            pltpu.sync_copy(src_hbm.at[pl.ds(hbase, RPH)], srcv)
            pltpu.sync_copy(dst_hbm.at[pl.ds(hbase, RPH)], dstv)

            pltpu.async_copy(x_hbm.at[srcv.at[0]], rows_a, sem_a)

            def body(k, carry):
                j0 = 2 * k
                j1 = j0 + 1
                pltpu.async_copy(x_hbm.at[srcv.at[j1]], rows_b, sem_b)
                pltpu.make_async_copy(
                    x_hbm.at[srcv.at[j0]], rows_a, sem_a).wait()
                pltpu.sync_copy(rows_a, agg_sh.at[dstv.at[j0]], add=True)

                @pl.when(k < RPH // 2 - 1)
                def _():
                    pltpu.async_copy(x_hbm.at[srcv.at[j0 + 2]], rows_a, sem_a)

                pltpu.make_async_copy(
                    x_hbm.at[srcv.at[j1]], rows_b, sem_b).wait()
                pltpu.sync_copy(rows_b, agg_sh.at[dstv.at[j1]], add=True)
                return carry

            lax.fori_loop(0, RPH // 2, body, 0)

        plsc.subcore_barrier()
        pltpu.sync_copy(
            agg_sh.at[pl.ds(sid * ROWS_PER_TILE, ROWS_PER_TILE)],
            out_hbm.at[cid, pl.ds(sid * ROWS_PER_TILE, ROWS_PER_TILE)])

    return agg_kernel(x, src, dst, zrows)


BN = 2000


def _tc_finish(parts, x, W):
    def body(p_ref, x_ref, w_ref, o_ref):
        agg = p_ref[0] + p_ref[1]
        h = lax.dot_general(agg, w_ref[...], (((1,), (1,)), ((), ())),
                            preferred_element_type=jnp.float32)
        o_ref[...] = jnp.maximum(h, 0.0) + x_ref[...]

    return pl.pallas_call(
        body,
        grid=(N_NODES // BN,),
        in_specs=[
            pl.BlockSpec((NC, BN, D), lambda i: (0, i, 0)),
            pl.BlockSpec((BN, D), lambda i: (i, 0)),
            pl.BlockSpec((D, D), lambda i: (0, 0)),
        ],
        out_specs=pl.BlockSpec((BN, D), lambda i: (i, 0)),
        out_shape=jax.ShapeDtypeStruct((N_NODES, D), jnp.float32),
    )(parts, x, W)


def kernel(x, edge_index, W):
    src = edge_index[0].astype(jnp.int32)
    dst = edge_index[1].astype(jnp.int32)
    pad = EPAD + EXTRA - N_EDGES
    arange_pad = jnp.arange(pad, dtype=jnp.int32)
    src_p = jnp.concatenate(
        [src, N_NODES + arange_pad % (NPAD - N_NODES)]).reshape(-1, CHUNK)
    dst_p = jnp.concatenate([dst, arange_pad % NPAD]).reshape(-1, CHUNK)
    x_g = jnp.concatenate(
        [x, jnp.zeros((NPAD - N_NODES, D), jnp.float32)])
    zrows = jnp.zeros((ROWS_PER_TILE, D), jnp.float32)
    parts = _sc_aggregate(x_g, src_p, dst_p, zrows)
    return _tc_finish(parts, x, W)
